# baseline (speedup 1.0000x reference)
import functools
import math

import jax
import jax.numpy as jnp
from jax import lax
from jax.experimental import pallas as pl
from jax.experimental.pallas import tpu as pltpu
from jax.experimental.pallas import tpu_sc as plsc

N = 10000
E = 320000
DH = 144
DA = 128
FEAT = 32
NC = 2
NS = 16
NW = NC * NS

SQRT3 = math.sqrt(3.0)
SQRT15 = math.sqrt(15.0)
C20 = math.sqrt(5.0) / 2.0
C22 = math.sqrt(15.0) / 2.0
EMB_SCALE = 1.14136 * math.exp(2.0) * math.sqrt(10.0)
STEP = 6.0 / 11.0
INV32 = 1.0 / math.sqrt(32.0)


_MESH = plsc.VectorSubcoreMesh(core_axis_name="c", subcore_axis_name="s")


def _make_gather(n_idx, C=80, LR=200):
    EW = n_idx // NW
    NCH = EW // C
    NLCH = N // LR

    @functools.partial(
        pl.kernel,
        out_type=jax.ShapeDtypeStruct((n_idx, DA), jnp.float32),
        mesh=_MESH,
        scratch_types=[
            pltpu.VMEM((C,), jnp.int32),
            pltpu.VMEM((C, DA), jnp.float32),
            pltpu.VMEM_SHARED((N, DA), jnp.float32),
            pltpu.SemaphoreType.DMA,
        ],
    )
    def gather_k(table, idx, out, idx_v, rows_v, tab, sem):
        c = lax.axis_index("c")
        s = lax.axis_index("s")

        def load(r, carry):
            ch = s + NS * r

            @pl.when(ch < NLCH)
            def _():
                pltpu.sync_copy(table.at[pl.ds(ch * LR, LR)],
                                tab.at[pl.ds(ch * LR, LR)])
            return carry

        lax.fori_loop(0, (NLCH + NS - 1) // NS, load, 0)
        plsc.subcore_barrier()

        base = (s * NC + c) * EW

        def body(j, carry):
            off = base + j * C
            pltpu.sync_copy(idx.at[pl.ds(off, C)], idx_v)
            pltpu.async_copy(tab.at[idx_v], rows_v, sem).wait()
            pltpu.sync_copy(rows_v, out.at[pl.ds(off, C)])
            return carry

        lax.fori_loop(0, NCH, body, 0)

    return gather_k


def _make_scatter(C=80, ZR=200):
    EW = E // NW
    NCH = EW // C
    NZCH = N // ZR

    @functools.partial(
        pl.kernel,
        out_type=jax.ShapeDtypeStruct((NC, N, DA), jnp.float32),
        mesh=_MESH,
        scratch_types=[
            pltpu.VMEM((C,), jnp.int32),
            pltpu.VMEM((C, DA), jnp.float32),
            pltpu.VMEM((ZR, DA), jnp.float32),
            pltpu.VMEM_SHARED((N, DA), jnp.float32),
            pltpu.SemaphoreType.DMA,
        ],
    )
    def scatter_k(msg, idx, out, idx_v, msg_v, zb, acc, sem):
        c = lax.axis_index("c")
        s = lax.axis_index("s")

        nvec = ZR * DA // 16

        def zvec(t, carry):
            zb[t // (DA // 16), pl.ds((t % (DA // 16)) * 16, 16)] = jnp.zeros(
                (16,), jnp.float32)
            return carry

        lax.fori_loop(0, nvec, zvec, 0)

        def zacc(r, carry):
            ch = s + NS * r

            @pl.when(ch < NZCH)
            def _():
                pltpu.sync_copy(zb, acc.at[pl.ds(ch * ZR, ZR)])
            return carry

        lax.fori_loop(0, (NZCH + NS - 1) // NS, zacc, 0)
        plsc.subcore_barrier()

        base = (s * NC + c) * EW

        def body(j, carry):
            off = base + j * C
            pltpu.sync_copy(idx.at[pl.ds(off, C)], idx_v)
            pltpu.sync_copy(msg.at[pl.ds(off, C)], msg_v)
            pltpu.sync_copy(msg_v, acc.at[idx_v], add=True)
            return carry

        lax.fori_loop(0, NCH, body, 0)
        plsc.subcore_barrier()

        def wout(r, carry):
            ch = s + NS * r

            @pl.when(ch < NZCH)
            def _():
                pltpu.sync_copy(acc.at[pl.ds(ch * ZR, ZR)],
                                out.at[c, pl.ds(ch * ZR, ZR)])
            return carry

        lax.fori_loop(0, (NZCH + NS - 1) // NS, wout, 0)

    return scatter_k


def _make_posdiff(C=80):
    EW = E // NW
    NCH = EW // C

    @functools.partial(
        pl.kernel,
        out_type=jax.ShapeDtypeStruct((16 * E,), jnp.float32),
        mesh=_MESH,
        compiler_params=pltpu.CompilerParams(needs_layout_passes=False),
        scratch_types=[
            pltpu.VMEM((4 * N,), jnp.float32),
            pltpu.VMEM((C,), jnp.int32),
            pltpu.VMEM((C,), jnp.int32),
            pltpu.VMEM((16 * C,), jnp.float32),
        ],
    )
    def posdiff_k(pos4, src, dst, out, tab, src_v, dst_v, stage):
        c = lax.axis_index("c")
        s = lax.axis_index("s")
        pltpu.sync_copy(pos4, tab)
        lane = lax.broadcasted_iota(jnp.int32, (16,), 0)
        zero16 = jnp.zeros((16,), jnp.float32)

        def zstage(t, carry):
            stage[pl.ds(t * 16, 16)] = zero16
            return carry

        lax.fori_loop(0, C, zstage, 0)
        base = (s * NC + c) * EW

        def body(j, carry):
            off = base + j * C
            pltpu.sync_copy(src.at[pl.ds(off, C)], src_v)
            pltpu.sync_copy(dst.at[pl.ds(off, C)], dst_v)

            def grp(k, carry2):
                si = src_v[pl.ds(k * 16, 16)] * 4
                di = dst_v[pl.ds(k * 16, 16)] * 4
                ebase = k * 256 + lane * 16
                for d in range(3):
                    ps = plsc.load_gather(tab, [si + d])
                    pd = plsc.load_gather(tab, [di + d])
                    plsc.store_scatter(stage, [ebase + d], ps - pd)
                return carry2

            lax.fori_loop(0, C // 16, grp, 0)
            pltpu.sync_copy(stage, out.at[pl.ds(off * 16, C * 16)])
            return carry

        lax.fori_loop(0, NCH, body, 0)

    return posdiff_k


def _make_gather_res(C=80):
    EW = E // NW
    NCH = EW // C

    @functools.partial(
        pl.kernel,
        out_type=[jax.ShapeDtypeStruct((8 * E,), jnp.float32),
                  jax.ShapeDtypeStruct((8 * E,), jnp.float32)],
        mesh=_MESH,
        compiler_params=pltpu.CompilerParams(needs_layout_passes=False),
        scratch_types=[
            pltpu.VMEM((8 * N,), jnp.float32),
            pltpu.VMEM((C,), jnp.int32),
            pltpu.VMEM((8 * C,), jnp.float32),
        ],
    )
    def gres_k(t1, t2, src, out1, out2, tab, src_v, stage):
        c = lax.axis_index("c")
        s = lax.axis_index("s")
        base = (s * NC + c) * EW
        lane = lax.broadcasted_iota(jnp.int32, (16,), 0)

        def one_pass(tsrc, out):
            pltpu.sync_copy(tsrc, tab)

            def body(j, carry):
                off = base + j * C
                pltpu.sync_copy(src.at[pl.ds(off, C)], src_v)

                def grp(k, carry2):
                    si = src_v[pl.ds(k * 16, 16)] * 8
                    ebase = k * 128 + lane * 8
                    for d in range(8):
                        v = plsc.load_gather(tab, [si + d])
                        plsc.store_scatter(stage, [ebase + d], v)
                    return carry2

                lax.fori_loop(0, C // 16, grp, 0)
                pltpu.sync_copy(stage, out.at[pl.ds(off * 8, C * 8)])
                return carry

            lax.fori_loop(0, NCH, body, 0)

        one_pass(t1, out1)
        one_pass(t2, out2)

    return gres_k


_posdiff = _make_posdiff()
_gather_e = _make_gather(E)
_gather_res = _make_gather_res()
_scatter = _make_scatter()


TE = 2560
TN = 1000


def _sus(t):
    safe = jnp.where(t > 0, t, 1.0)
    return jnp.where(t > 0, jnp.exp(-1.0 / safe), 0.0)


def _edge_feat_body(dv_ref, attr_ref, feat_ref):
    dv = dv_ref[:, :3]
    ln = jnp.sqrt(jnp.sum(dv * dv, axis=1, keepdims=True) + 1e-12)
    u = dv / ln
    ux, uy, uz = u[:, 0:1], u[:, 1:2], u[:, 2:3]
    sh = jnp.concatenate([
        jnp.ones_like(ux), SQRT3 * ux, SQRT3 * uy, SQRT3 * uz,
        SQRT15 * ux * uy, SQRT15 * uy * uz, C20 * (3.0 * uz * uz - 1.0),
        SQRT15 * ux * uz, C22 * (ux * ux - uy * uy)], axis=1)
    vals = (lax.broadcasted_iota(jnp.int32, (1, 10), 1).astype(jnp.float32)
            + 1.0) * STEP
    diff = (ln - vals) / STEP
    emb = EMB_SCALE * _sus(diff + 1.0) * _sus(1.0 - diff)
    z = jnp.zeros((dv_ref.shape[0], 1), jnp.float32)
    feat_ref[...] = jnp.concatenate(
        [attr_ref[...], sh, z, z, z, emb, z, z, z, z, z, z], axis=1)


def _edge_feat(gdv, edge_attr):
    grid = E // TE
    return pl.pallas_call(
        _edge_feat_body,
        grid=(grid,),
        in_specs=[
            pl.BlockSpec((TE, 16), lambda i: (i, 0)),
            pl.BlockSpec((TE, 4), lambda i: (i, 0)),
        ],
        out_specs=pl.BlockSpec((TE, FEAT), lambda i: (i, 0)),
        out_shape=jax.ShapeDtypeStruct((E, FEAT), jnp.float32),
    )(gdv, edge_attr)


def _combine1_body(feat_ref, ga_ref, weaa_ref, wf1_ref, wf2a_ref, msga_ref):
    f = feat_ref[...]
    gl = jax.nn.gelu(jnp.dot(f, wf1_ref[...],
                             preferred_element_type=jnp.float32))
    wea = jnp.dot(gl, wf2a_ref[...], preferred_element_type=jnp.float32)
    eaa = jnp.dot(f, weaa_ref[...], preferred_element_type=jnp.float32)
    msga_ref[...] = (ga_ref[...] + eaa) * wea


def _combine1(feat, ga, weaa, wf1, wf2a):
    grid = E // TE
    return pl.pallas_call(
        _combine1_body,
        grid=(grid,),
        in_specs=[
            pl.BlockSpec((TE, FEAT), lambda i: (i, 0)),
            pl.BlockSpec((TE, DA), lambda i: (i, 0)),
            pl.BlockSpec((FEAT, DA), lambda i: (0, 0)),
            pl.BlockSpec((FEAT, 100), lambda i: (0, 0)),
            pl.BlockSpec((100, DA), lambda i: (0, 0)),
        ],
        out_specs=pl.BlockSpec((TE, DA), lambda i: (i, 0)),
        out_shape=jax.ShapeDtypeStruct((E, DA), jnp.float32),
    )(feat, ga, weaa, wf1, wf2a)


def _combineb_body(feat_ref, gb1_ref, gb2_ref, weab_ref, wf1_ref, wf2b_ref,
                   msgb_ref):
    f = feat_ref[...]
    gl = jax.nn.gelu(jnp.dot(f, wf1_ref[...],
                             preferred_element_type=jnp.float32))
    web = jnp.dot(gl, wf2b_ref[...], preferred_element_type=jnp.float32)
    eab = jnp.dot(f, weab_ref[...], preferred_element_type=jnp.float32)
    gb = jnp.concatenate([gb1_ref[...], gb2_ref[...]], axis=1)
    m = (gb + eab) * web
    msgb_ref[...] = jnp.concatenate(
        [m, jnp.zeros((m.shape[0], DA - 16), jnp.float32)], axis=1)


def _combineb(feat, gb1, gb2, weab, wf1, wf2b):
    grid = E // TE
    return pl.pallas_call(
        _combineb_body,
        grid=(grid,),
        in_specs=[
            pl.BlockSpec((TE, FEAT), lambda i: (i, 0)),
            pl.BlockSpec((TE, 8), lambda i: (i, 0)),
            pl.BlockSpec((TE, 8), lambda i: (i, 0)),
            pl.BlockSpec((FEAT, 16), lambda i: (0, 0)),
            pl.BlockSpec((FEAT, 100), lambda i: (0, 0)),
            pl.BlockSpec((100, 16), lambda i: (0, 0)),
        ],
        out_specs=pl.BlockSpec((TE, DA), lambda i: (i, 0)),
        out_shape=jax.ShapeDtypeStruct((E, DA), jnp.float32),
    )(feat, gb1, gb2, weab, wf1, wf2b)


def _pre0_body(x_ref, wa_ref, wb1_ref, wb2_ref, hwa_ref, hwb1_ref, hwb2_ref):
    x = x_ref[...]
    hwa_ref[...] = jnp.dot(x, wa_ref[...], preferred_element_type=jnp.float32)
    hwb1_ref[...] = jnp.dot(x, wb1_ref[...], preferred_element_type=jnp.float32)
    hwb2_ref[...] = jnp.dot(x, wb2_ref[...], preferred_element_type=jnp.float32)


def _pre0(x, wa, wb1, wb2):
    return pl.pallas_call(
        _pre0_body,
        grid=(N // TN,),
        in_specs=[
            pl.BlockSpec((TN, 8), lambda i: (i, 0)),
            pl.BlockSpec((8, DA), lambda i: (0, 0)),
            pl.BlockSpec((8, 8), lambda i: (0, 0)),
            pl.BlockSpec((8, 8), lambda i: (0, 0)),
        ],
        out_specs=[
            pl.BlockSpec((TN, DA), lambda i: (i, 0)),
            pl.BlockSpec((TN, 8), lambda i: (i, 0)),
            pl.BlockSpec((TN, 8), lambda i: (i, 0)),
        ],
        out_shape=[
            jax.ShapeDtypeStruct((N, DA), jnp.float32),
            jax.ShapeDtypeStruct((N, 8), jnp.float32),
            jax.ShapeDtypeStruct((N, 8), jnp.float32),
        ],
    )(x, wa, wb1, wb2)


def _upd_body(aa0_ref, aa1_ref, ab0_ref, ab1_ref, h_ref, na_ref, ws_ref,
              wta_ref, wtb1_ref, wtb2_ref, hout_ref, hwa_ref, hwb1_ref,
              hwb2_ref):
    agga = (aa0_ref[0] + aa1_ref[0]) * INV32
    aggb = ((ab0_ref[0] + ab1_ref[0]) * INV32)[:, :DH - DA]
    agg = jnp.concatenate([agga, aggb], axis=1)
    hn = agg + jnp.dot(h_ref[...], ws_ref[...],
                       preferred_element_type=jnp.float32)
    hn = jax.nn.gelu(hn * na_ref[...])
    hout_ref[...] = hn
    hwa_ref[...] = jnp.dot(hn, wta_ref[...], preferred_element_type=jnp.float32)
    hwb1_ref[...] = jnp.dot(hn, wtb1_ref[...],
                            preferred_element_type=jnp.float32)
    hwb2_ref[...] = jnp.dot(hn, wtb2_ref[...],
                            preferred_element_type=jnp.float32)


def _upd(agga, aggb, h, node_attr, wself, wta, wtb1, wtb2, di):
    return pl.pallas_call(
        _upd_body,
        grid=(N // TN,),
        in_specs=[
            pl.BlockSpec((1, TN, DA), lambda i: (0, i, 0)),
            pl.BlockSpec((1, TN, DA), lambda i: (1, i, 0)),
            pl.BlockSpec((1, TN, DA), lambda i: (0, i, 0)),
            pl.BlockSpec((1, TN, DA), lambda i: (1, i, 0)),
            pl.BlockSpec((TN, di), lambda i: (i, 0)),
            pl.BlockSpec((TN, 1), lambda i: (i, 0)),
            pl.BlockSpec((di, DH), lambda i: (0, 0)),
            pl.BlockSpec((DH, DA), lambda i: (0, 0)),
            pl.BlockSpec((DH, 8), lambda i: (0, 0)),
            pl.BlockSpec((DH, 8), lambda i: (0, 0)),
        ],
        out_specs=[
            pl.BlockSpec((TN, DH), lambda i: (i, 0)),
            pl.BlockSpec((TN, DA), lambda i: (i, 0)),
            pl.BlockSpec((TN, 8), lambda i: (i, 0)),
            pl.BlockSpec((TN, 8), lambda i: (i, 0)),
        ],
        out_shape=[
            jax.ShapeDtypeStruct((N, DH), jnp.float32),
            jax.ShapeDtypeStruct((N, DA), jnp.float32),
            jax.ShapeDtypeStruct((N, 8), jnp.float32),
            jax.ShapeDtypeStruct((N, 8), jnp.float32),
        ],
    )(agga, agga, aggb, aggb, h, node_attr, wself, wta, wtb1, wtb2)


def _fin_body(aa0_ref, aa1_ref, h_ref, na_ref, ws_ref, out_ref):
    i = pl.program_id(0)
    agg = (aa0_ref[0][:, :3] + aa1_ref[0][:, :3]) * INV32
    hv = (agg + jnp.dot(h_ref[...], ws_ref[...],
                        preferred_element_type=jnp.float32)) * na_ref[...]
    part = jnp.sum(hv, axis=0, keepdims=True) * 0.01

    @pl.when(i == 0)
    def _():
        out_ref[...] = part

    @pl.when(i > 0)
    def _():
        out_ref[...] += part


def _fin(agga, h, node_attr, wself3):
    return pl.pallas_call(
        _fin_body,
        grid=(N // TN,),
        in_specs=[
            pl.BlockSpec((1, TN, DA), lambda i: (0, i, 0)),
            pl.BlockSpec((1, TN, DA), lambda i: (1, i, 0)),
            pl.BlockSpec((TN, DH), lambda i: (i, 0)),
            pl.BlockSpec((TN, 1), lambda i: (i, 0)),
            pl.BlockSpec((DH, 3), lambda i: (0, 0)),
        ],
        out_specs=pl.BlockSpec((1, 3), lambda i: (0, 0)),
        out_shape=jax.ShapeDtypeStruct((1, 3), jnp.float32),
    )(agga, agga, h, node_attr, wself3)


def _pad_cols(a, w):
    return jnp.pad(a, ((0, 0), (0, w - a.shape[1])))


def kernel(pos, x, node_attr, edge_attr, edge_src, edge_dst, batch,
           W_msg_0, W_fc1_0, W_fc2_0, W_self_0,
           W_msg_1, W_fc1_1, W_fc2_1, W_self_1,
           W_msg_2, W_fc1_2, W_fc2_2, W_self_2,
           W_msg_3, W_fc1_3, W_fc2_3, W_self_3):
    Wm = [W_msg_0, W_msg_1, W_msg_2, W_msg_3]
    Wf1 = [W_fc1_0, W_fc1_1, W_fc1_2, W_fc1_3]
    Wf2 = [W_fc2_0, W_fc2_1, W_fc2_2, W_fc2_3]
    Ws = [W_self_0, W_self_1, W_self_2, W_self_3]
    dims = [8, DH, DH, DH]

    wtopa, wtb1, wtb2 = [], [], []
    weaa_p, weab16, wf1_p, wf2a_p, wf2b16 = [], [], [], [], []
    for i in range(4):
        di = dims[i]
        wtop_i = Wm[i][:di]
        wea_i = Wm[i][di:]
        do = Wm[i].shape[1]
        wtopa.append(_pad_cols(wtop_i[:, :min(do, DA)], DA))
        wea_full = jnp.concatenate(
            [wea_i, jnp.zeros((FEAT - 13, do), jnp.float32)], axis=0)
        weaa_p.append(_pad_cols(wea_full[:, :min(do, DA)], DA))
        wf1_p.append(jnp.concatenate([
            jnp.zeros((16, 100), jnp.float32), Wf1[i],
            jnp.zeros((6, 100), jnp.float32)], axis=0))
        wf2a_p.append(_pad_cols(Wf2[i][:, :min(do, DA)], DA))
        if do > DA:
            wtb1.append(wtop_i[:, DA:DA + 8])
            wtb2.append(wtop_i[:, DA + 8:])
            weab16.append(wea_full[:, DA:])
            wf2b16.append(Wf2[i][:, DA:])
        else:
            wtb1.append(jnp.zeros((di, 8), jnp.float32))
            wtb2.append(jnp.zeros((di, 8), jnp.float32))
            weab16.append(None)
            wf2b16.append(None)

    pos4 = _pad_cols(pos, 4).reshape(-1)
    gdv = _posdiff(pos4, edge_src, edge_dst).reshape(E, 16)
    feat = _edge_feat(gdv, edge_attr)

    hwa, hwb1, hwb2 = _pre0(x, wtopa[0], wtb1[0], wtb2[0])
    h = x
    for i in range(3):
        ga = _gather_e(hwa, edge_src)
        grb1, grb2 = _gather_res(hwb1.reshape(-1), hwb2.reshape(-1),
                                 edge_src)
        msga = _combine1(feat, ga, weaa_p[i], wf1_p[i], wf2a_p[i])
        agga = _scatter(msga, edge_dst)
        msgb = _combineb(feat, grb1.reshape(E, 8), grb2.reshape(E, 8),
                         weab16[i], wf1_p[i], wf2b16[i])
        dst2, _ = lax.optimization_barrier((edge_dst, agga))
        aggb = _scatter(msgb, dst2)
        h, hwa, hwb1, hwb2 = _upd(agga, aggb, h, node_attr, Ws[i],
                                  wtopa[i + 1], wtb1[i + 1], wtb2[i + 1],
                                  dims[i])
    ga = _gather_e(hwa, edge_src)
    msga = _combine1(feat, ga, weaa_p[3], wf1_p[3], wf2a_p[3])
    agga = _scatter(msga, edge_dst)
    return _fin(agga, h, node_attr, Ws[3])


# speedup vs baseline: 1.0282x; 1.0282x over previous
import functools
import math

import jax
import jax.numpy as jnp
from jax import lax
from jax.experimental import pallas as pl
from jax.experimental.pallas import tpu as pltpu
from jax.experimental.pallas import tpu_sc as plsc

N = 10000
E = 320000
DH = 144
DA = 128
FEAT = 32
NC = 2
NS = 16
NW = NC * NS

SQRT3 = math.sqrt(3.0)
SQRT15 = math.sqrt(15.0)
C20 = math.sqrt(5.0) / 2.0
C22 = math.sqrt(15.0) / 2.0
EMB_SCALE = 1.14136 * math.exp(2.0) * math.sqrt(10.0)
STEP = 6.0 / 11.0
INV32 = 1.0 / math.sqrt(32.0)


_MESH = plsc.VectorSubcoreMesh(core_axis_name="c", subcore_axis_name="s")


def _make_gather(n_idx, C=80, LR=200):
    EW = n_idx // NW
    NCH = EW // C
    NLCH = N // LR

    @functools.partial(
        pl.kernel,
        out_type=jax.ShapeDtypeStruct((n_idx, DA), jnp.float32),
        mesh=_MESH,
        scratch_types=[
            pltpu.VMEM((C,), jnp.int32),
            pltpu.VMEM((C, DA), jnp.float32),
            pltpu.VMEM_SHARED((N, DA), jnp.float32),
            pltpu.SemaphoreType.DMA,
        ],
    )
    def gather_k(table, idx, out, idx_v, rows_v, tab, sem):
        c = lax.axis_index("c")
        s = lax.axis_index("s")

        def load(r, carry):
            ch = s + NS * r

            @pl.when(ch < NLCH)
            def _():
                pltpu.sync_copy(table.at[pl.ds(ch * LR, LR)],
                                tab.at[pl.ds(ch * LR, LR)])
            return carry

        lax.fori_loop(0, (NLCH + NS - 1) // NS, load, 0)
        plsc.subcore_barrier()

        base = (s * NC + c) * EW

        def body(j, carry):
            off = base + j * C
            pltpu.sync_copy(idx.at[pl.ds(off, C)], idx_v)
            pltpu.async_copy(tab.at[idx_v], rows_v, sem).wait()
            pltpu.sync_copy(rows_v, out.at[pl.ds(off, C)])
            return carry

        lax.fori_loop(0, NCH, body, 0)

    return gather_k


def _make_scatter(C=80, ZR=200):
    EW = E // NW
    NCH = EW // C
    NZCH = N // ZR

    @functools.partial(
        pl.kernel,
        out_type=jax.ShapeDtypeStruct((NC, N, DA), jnp.float32),
        mesh=_MESH,
        scratch_types=[
            pltpu.VMEM((C,), jnp.int32),
            pltpu.VMEM((C, DA), jnp.float32),
            pltpu.VMEM((ZR, DA), jnp.float32),
            pltpu.VMEM_SHARED((N, DA), jnp.float32),
            pltpu.SemaphoreType.DMA,
        ],
    )
    def scatter_k(msg, idx, out, idx_v, msg_v, zb, acc, sem):
        c = lax.axis_index("c")
        s = lax.axis_index("s")

        nvec = ZR * DA // 16

        def zvec(t, carry):
            zb[t // (DA // 16), pl.ds((t % (DA // 16)) * 16, 16)] = jnp.zeros(
                (16,), jnp.float32)
            return carry

        lax.fori_loop(0, nvec, zvec, 0)

        def zacc(r, carry):
            ch = s + NS * r

            @pl.when(ch < NZCH)
            def _():
                pltpu.sync_copy(zb, acc.at[pl.ds(ch * ZR, ZR)])
            return carry

        lax.fori_loop(0, (NZCH + NS - 1) // NS, zacc, 0)
        plsc.subcore_barrier()

        base = (s * NC + c) * EW

        def body(j, carry):
            off = base + j * C
            pltpu.sync_copy(idx.at[pl.ds(off, C)], idx_v)
            pltpu.sync_copy(msg.at[pl.ds(off, C)], msg_v)
            pltpu.sync_copy(msg_v, acc.at[idx_v], add=True)
            return carry

        lax.fori_loop(0, NCH, body, 0)
        plsc.subcore_barrier()

        def wout(r, carry):
            ch = s + NS * r

            @pl.when(ch < NZCH)
            def _():
                pltpu.sync_copy(acc.at[pl.ds(ch * ZR, ZR)],
                                out.at[c, pl.ds(ch * ZR, ZR)])
            return carry

        lax.fori_loop(0, (NZCH + NS - 1) // NS, wout, 0)

    return scatter_k


def _make_posdiff(C=2000):
    EW = E // NW
    NCH = EW // C

    @functools.partial(
        pl.kernel,
        out_type=jax.ShapeDtypeStruct((16 * E,), jnp.float32),
        mesh=_MESH,
        compiler_params=pltpu.CompilerParams(needs_layout_passes=False),
        scratch_types=[
            pltpu.VMEM((4 * N,), jnp.float32),
            pltpu.VMEM((C,), jnp.int32),
            pltpu.VMEM((C,), jnp.int32),
            pltpu.VMEM((16 * C,), jnp.float32),
        ],
    )
    def posdiff_k(pos4, src, dst, out, tab, src_v, dst_v, stage):
        c = lax.axis_index("c")
        s = lax.axis_index("s")
        pltpu.sync_copy(pos4, tab)
        lane = lax.broadcasted_iota(jnp.int32, (16,), 0)
        zero16 = jnp.zeros((16,), jnp.float32)

        def zstage(t, carry):
            stage[pl.ds(t * 16, 16)] = zero16
            return carry

        lax.fori_loop(0, C, zstage, 0)
        base = (s * NC + c) * EW

        def body(j, carry):
            off = base + j * C
            pltpu.sync_copy(src.at[pl.ds(off, C)], src_v)
            pltpu.sync_copy(dst.at[pl.ds(off, C)], dst_v)

            def grp(k, carry2):
                si = src_v[pl.ds(k * 16, 16)] * 4
                di = dst_v[pl.ds(k * 16, 16)] * 4
                ebase = k * 256 + lane * 16
                for d in range(3):
                    ps = plsc.load_gather(tab, [si + d])
                    pd = plsc.load_gather(tab, [di + d])
                    plsc.store_scatter(stage, [ebase + d], ps - pd)
                return carry2

            lax.fori_loop(0, C // 16, grp, 0)
            pltpu.sync_copy(stage, out.at[pl.ds(off * 16, C * 16)])
            return carry

        lax.fori_loop(0, NCH, body, 0)

    return posdiff_k


def _make_gather_res(C=2000):
    EW = E // NW
    NCH = EW // C

    @functools.partial(
        pl.kernel,
        out_type=[jax.ShapeDtypeStruct((8 * E,), jnp.float32),
                  jax.ShapeDtypeStruct((8 * E,), jnp.float32)],
        mesh=_MESH,
        compiler_params=pltpu.CompilerParams(needs_layout_passes=False),
        scratch_types=[
            pltpu.VMEM((8 * N,), jnp.float32),
            pltpu.VMEM((C,), jnp.int32),
            pltpu.VMEM((8 * C,), jnp.float32),
        ],
    )
    def gres_k(t1, t2, src, out1, out2, tab, src_v, stage):
        c = lax.axis_index("c")
        s = lax.axis_index("s")
        base = (s * NC + c) * EW
        lane = lax.broadcasted_iota(jnp.int32, (16,), 0)

        def one_pass(tsrc, out):
            pltpu.sync_copy(tsrc, tab)

            def body(j, carry):
                off = base + j * C
                pltpu.sync_copy(src.at[pl.ds(off, C)], src_v)

                def grp(k, carry2):
                    si = src_v[pl.ds(k * 16, 16)] * 8
                    ebase = k * 128 + lane * 8
                    for d in range(8):
                        v = plsc.load_gather(tab, [si + d])
                        plsc.store_scatter(stage, [ebase + d], v)
                    return carry2

                lax.fori_loop(0, C // 16, grp, 0)
                pltpu.sync_copy(stage, out.at[pl.ds(off * 8, C * 8)])
                return carry

            lax.fori_loop(0, NCH, body, 0)

        one_pass(t1, out1)
        one_pass(t2, out2)

    return gres_k


_posdiff = _make_posdiff()
_gather_e = _make_gather(E)
_gather_res = _make_gather_res()
_scatter = _make_scatter()


TE = 2560
TN = 1000


def _sus(t):
    safe = jnp.where(t > 0, t, 1.0)
    return jnp.where(t > 0, jnp.exp(-1.0 / safe), 0.0)


def _edge_feat_body(dv_ref, attr_ref, feat_ref):
    dv = dv_ref[:, :3]
    ln = jnp.sqrt(jnp.sum(dv * dv, axis=1, keepdims=True) + 1e-12)
    u = dv / ln
    ux, uy, uz = u[:, 0:1], u[:, 1:2], u[:, 2:3]
    sh = jnp.concatenate([
        jnp.ones_like(ux), SQRT3 * ux, SQRT3 * uy, SQRT3 * uz,
        SQRT15 * ux * uy, SQRT15 * uy * uz, C20 * (3.0 * uz * uz - 1.0),
        SQRT15 * ux * uz, C22 * (ux * ux - uy * uy)], axis=1)
    vals = (lax.broadcasted_iota(jnp.int32, (1, 10), 1).astype(jnp.float32)
            + 1.0) * STEP
    diff = (ln - vals) / STEP
    emb = EMB_SCALE * _sus(diff + 1.0) * _sus(1.0 - diff)
    z = jnp.zeros((dv_ref.shape[0], 1), jnp.float32)
    feat_ref[...] = jnp.concatenate(
        [attr_ref[...], sh, z, z, z, emb, z, z, z, z, z, z], axis=1)


def _edge_feat(gdv, edge_attr):
    grid = E // TE
    return pl.pallas_call(
        _edge_feat_body,
        grid=(grid,),
        in_specs=[
            pl.BlockSpec((TE, 16), lambda i: (i, 0)),
            pl.BlockSpec((TE, 4), lambda i: (i, 0)),
        ],
        out_specs=pl.BlockSpec((TE, FEAT), lambda i: (i, 0)),
        out_shape=jax.ShapeDtypeStruct((E, FEAT), jnp.float32),
    )(gdv, edge_attr)


def _combine1_body(feat_ref, ga_ref, weaa_ref, wf1_ref, wf2a_ref, msga_ref):
    f = feat_ref[...]
    gl = jax.nn.gelu(jnp.dot(f, wf1_ref[...],
                             preferred_element_type=jnp.float32))
    wea = jnp.dot(gl, wf2a_ref[...], preferred_element_type=jnp.float32)
    eaa = jnp.dot(f, weaa_ref[...], preferred_element_type=jnp.float32)
    msga_ref[...] = (ga_ref[...] + eaa) * wea


def _combine1(feat, ga, weaa, wf1, wf2a):
    grid = E // TE
    return pl.pallas_call(
        _combine1_body,
        grid=(grid,),
        in_specs=[
            pl.BlockSpec((TE, FEAT), lambda i: (i, 0)),
            pl.BlockSpec((TE, DA), lambda i: (i, 0)),
            pl.BlockSpec((FEAT, DA), lambda i: (0, 0)),
            pl.BlockSpec((FEAT, 100), lambda i: (0, 0)),
            pl.BlockSpec((100, DA), lambda i: (0, 0)),
        ],
        out_specs=pl.BlockSpec((TE, DA), lambda i: (i, 0)),
        out_shape=jax.ShapeDtypeStruct((E, DA), jnp.float32),
    )(feat, ga, weaa, wf1, wf2a)


def _combineb_body(feat_ref, gb1_ref, gb2_ref, weab_ref, wf1_ref, wf2b_ref,
                   msgb_ref):
    f = feat_ref[...]
    gl = jax.nn.gelu(jnp.dot(f, wf1_ref[...],
                             preferred_element_type=jnp.float32))
    web = jnp.dot(gl, wf2b_ref[...], preferred_element_type=jnp.float32)
    eab = jnp.dot(f, weab_ref[...], preferred_element_type=jnp.float32)
    gb = jnp.concatenate([gb1_ref[...], gb2_ref[...]], axis=1)
    m = (gb + eab) * web
    msgb_ref[...] = jnp.concatenate(
        [m, jnp.zeros((m.shape[0], DA - 16), jnp.float32)], axis=1)


def _combineb(feat, gb1, gb2, weab, wf1, wf2b):
    grid = E // TE
    return pl.pallas_call(
        _combineb_body,
        grid=(grid,),
        in_specs=[
            pl.BlockSpec((TE, FEAT), lambda i: (i, 0)),
            pl.BlockSpec((TE, 8), lambda i: (i, 0)),
            pl.BlockSpec((TE, 8), lambda i: (i, 0)),
            pl.BlockSpec((FEAT, 16), lambda i: (0, 0)),
            pl.BlockSpec((FEAT, 100), lambda i: (0, 0)),
            pl.BlockSpec((100, 16), lambda i: (0, 0)),
        ],
        out_specs=pl.BlockSpec((TE, DA), lambda i: (i, 0)),
        out_shape=jax.ShapeDtypeStruct((E, DA), jnp.float32),
    )(feat, gb1, gb2, weab, wf1, wf2b)


def _pre0_body(x_ref, wa_ref, wb1_ref, wb2_ref, hwa_ref, hwb1_ref, hwb2_ref):
    x = x_ref[...]
    hwa_ref[...] = jnp.dot(x, wa_ref[...], preferred_element_type=jnp.float32)
    hwb1_ref[...] = jnp.dot(x, wb1_ref[...], preferred_element_type=jnp.float32)
    hwb2_ref[...] = jnp.dot(x, wb2_ref[...], preferred_element_type=jnp.float32)


def _pre0(x, wa, wb1, wb2):
    return pl.pallas_call(
        _pre0_body,
        grid=(N // TN,),
        in_specs=[
            pl.BlockSpec((TN, 8), lambda i: (i, 0)),
            pl.BlockSpec((8, DA), lambda i: (0, 0)),
            pl.BlockSpec((8, 8), lambda i: (0, 0)),
            pl.BlockSpec((8, 8), lambda i: (0, 0)),
        ],
        out_specs=[
            pl.BlockSpec((TN, DA), lambda i: (i, 0)),
            pl.BlockSpec((TN, 8), lambda i: (i, 0)),
            pl.BlockSpec((TN, 8), lambda i: (i, 0)),
        ],
        out_shape=[
            jax.ShapeDtypeStruct((N, DA), jnp.float32),
            jax.ShapeDtypeStruct((N, 8), jnp.float32),
            jax.ShapeDtypeStruct((N, 8), jnp.float32),
        ],
    )(x, wa, wb1, wb2)


def _upd_body(aa0_ref, aa1_ref, ab0_ref, ab1_ref, h_ref, na_ref, ws_ref,
              wta_ref, wtb1_ref, wtb2_ref, hout_ref, hwa_ref, hwb1_ref,
              hwb2_ref):
    agga = (aa0_ref[0] + aa1_ref[0]) * INV32
    aggb = ((ab0_ref[0] + ab1_ref[0]) * INV32)[:, :DH - DA]
    agg = jnp.concatenate([agga, aggb], axis=1)
    hn = agg + jnp.dot(h_ref[...], ws_ref[...],
                       preferred_element_type=jnp.float32)
    hn = jax.nn.gelu(hn * na_ref[...])
    hout_ref[...] = hn
    hwa_ref[...] = jnp.dot(hn, wta_ref[...], preferred_element_type=jnp.float32)
    hwb1_ref[...] = jnp.dot(hn, wtb1_ref[...],
                            preferred_element_type=jnp.float32)
    hwb2_ref[...] = jnp.dot(hn, wtb2_ref[...],
                            preferred_element_type=jnp.float32)


def _upd(agga, aggb, h, node_attr, wself, wta, wtb1, wtb2, di):
    return pl.pallas_call(
        _upd_body,
        grid=(N // TN,),
        in_specs=[
            pl.BlockSpec((1, TN, DA), lambda i: (0, i, 0)),
            pl.BlockSpec((1, TN, DA), lambda i: (1, i, 0)),
            pl.BlockSpec((1, TN, DA), lambda i: (0, i, 0)),
            pl.BlockSpec((1, TN, DA), lambda i: (1, i, 0)),
            pl.BlockSpec((TN, di), lambda i: (i, 0)),
            pl.BlockSpec((TN, 1), lambda i: (i, 0)),
            pl.BlockSpec((di, DH), lambda i: (0, 0)),
            pl.BlockSpec((DH, DA), lambda i: (0, 0)),
            pl.BlockSpec((DH, 8), lambda i: (0, 0)),
            pl.BlockSpec((DH, 8), lambda i: (0, 0)),
        ],
        out_specs=[
            pl.BlockSpec((TN, DH), lambda i: (i, 0)),
            pl.BlockSpec((TN, DA), lambda i: (i, 0)),
            pl.BlockSpec((TN, 8), lambda i: (i, 0)),
            pl.BlockSpec((TN, 8), lambda i: (i, 0)),
        ],
        out_shape=[
            jax.ShapeDtypeStruct((N, DH), jnp.float32),
            jax.ShapeDtypeStruct((N, DA), jnp.float32),
            jax.ShapeDtypeStruct((N, 8), jnp.float32),
            jax.ShapeDtypeStruct((N, 8), jnp.float32),
        ],
    )(agga, agga, aggb, aggb, h, node_attr, wself, wta, wtb1, wtb2)


def _fin_body(aa0_ref, aa1_ref, h_ref, na_ref, ws_ref, out_ref):
    i = pl.program_id(0)
    agg = (aa0_ref[0][:, :3] + aa1_ref[0][:, :3]) * INV32
    hv = (agg + jnp.dot(h_ref[...], ws_ref[...],
                        preferred_element_type=jnp.float32)) * na_ref[...]
    part = jnp.sum(hv, axis=0, keepdims=True) * 0.01

    @pl.when(i == 0)
    def _():
        out_ref[...] = part

    @pl.when(i > 0)
    def _():
        out_ref[...] += part


def _fin(agga, h, node_attr, wself3):
    return pl.pallas_call(
        _fin_body,
        grid=(N // TN,),
        in_specs=[
            pl.BlockSpec((1, TN, DA), lambda i: (0, i, 0)),
            pl.BlockSpec((1, TN, DA), lambda i: (1, i, 0)),
            pl.BlockSpec((TN, DH), lambda i: (i, 0)),
            pl.BlockSpec((TN, 1), lambda i: (i, 0)),
            pl.BlockSpec((DH, 3), lambda i: (0, 0)),
        ],
        out_specs=pl.BlockSpec((1, 3), lambda i: (0, 0)),
        out_shape=jax.ShapeDtypeStruct((1, 3), jnp.float32),
    )(agga, agga, h, node_attr, wself3)


def _pad_cols(a, w):
    return jnp.pad(a, ((0, 0), (0, w - a.shape[1])))


def kernel(pos, x, node_attr, edge_attr, edge_src, edge_dst, batch,
           W_msg_0, W_fc1_0, W_fc2_0, W_self_0,
           W_msg_1, W_fc1_1, W_fc2_1, W_self_1,
           W_msg_2, W_fc1_2, W_fc2_2, W_self_2,
           W_msg_3, W_fc1_3, W_fc2_3, W_self_3):
    Wm = [W_msg_0, W_msg_1, W_msg_2, W_msg_3]
    Wf1 = [W_fc1_0, W_fc1_1, W_fc1_2, W_fc1_3]
    Wf2 = [W_fc2_0, W_fc2_1, W_fc2_2, W_fc2_3]
    Ws = [W_self_0, W_self_1, W_self_2, W_self_3]
    dims = [8, DH, DH, DH]

    wtopa, wtb1, wtb2 = [], [], []
    weaa_p, weab16, wf1_p, wf2a_p, wf2b16 = [], [], [], [], []
    for i in range(4):
        di = dims[i]
        wtop_i = Wm[i][:di]
        wea_i = Wm[i][di:]
        do = Wm[i].shape[1]
        wtopa.append(_pad_cols(wtop_i[:, :min(do, DA)], DA))
        wea_full = jnp.concatenate(
            [wea_i, jnp.zeros((FEAT - 13, do), jnp.float32)], axis=0)
        weaa_p.append(_pad_cols(wea_full[:, :min(do, DA)], DA))
        wf1_p.append(jnp.concatenate([
            jnp.zeros((16, 100), jnp.float32), Wf1[i],
            jnp.zeros((6, 100), jnp.float32)], axis=0))
        wf2a_p.append(_pad_cols(Wf2[i][:, :min(do, DA)], DA))
        if do > DA:
            wtb1.append(wtop_i[:, DA:DA + 8])
            wtb2.append(wtop_i[:, DA + 8:])
            weab16.append(wea_full[:, DA:])
            wf2b16.append(Wf2[i][:, DA:])
        else:
            wtb1.append(jnp.zeros((di, 8), jnp.float32))
            wtb2.append(jnp.zeros((di, 8), jnp.float32))
            weab16.append(None)
            wf2b16.append(None)

    pos4 = _pad_cols(pos, 4).reshape(-1)
    gdv = _posdiff(pos4, edge_src, edge_dst).reshape(E, 16)
    feat = _edge_feat(gdv, edge_attr)

    hwa, hwb1, hwb2 = _pre0(x, wtopa[0], wtb1[0], wtb2[0])
    h = x
    for i in range(3):
        ga = _gather_e(hwa, edge_src)
        grb1, grb2 = _gather_res(hwb1.reshape(-1), hwb2.reshape(-1),
                                 edge_src)
        msga = _combine1(feat, ga, weaa_p[i], wf1_p[i], wf2a_p[i])
        agga = _scatter(msga, edge_dst)
        msgb = _combineb(feat, grb1.reshape(E, 8), grb2.reshape(E, 8),
                         weab16[i], wf1_p[i], wf2b16[i])
        dst2, _ = lax.optimization_barrier((edge_dst, agga))
        aggb = _scatter(msgb, dst2)
        h, hwa, hwb1, hwb2 = _upd(agga, aggb, h, node_attr, Ws[i],
                                  wtopa[i + 1], wtb1[i + 1], wtb2[i + 1],
                                  dims[i])
    ga = _gather_e(hwa, edge_src)
    msga = _combine1(feat, ga, weaa_p[3], wf1_p[3], wf2a_p[3])
    agga = _scatter(msga, edge_dst)
    return _fin(agga, h, node_attr, Ws[3])


# speedup vs baseline: 1.0826x; 1.0529x over previous
import functools
import math

import jax
import jax.numpy as jnp
from jax import lax
from jax.experimental import pallas as pl
from jax.experimental.pallas import tpu as pltpu
from jax.experimental.pallas import tpu_sc as plsc

N = 10000
E = 320000
DH = 144
DA = 128
FEAT = 32
NC = 2
NS = 16
NW = NC * NS

SQRT3 = math.sqrt(3.0)
SQRT15 = math.sqrt(15.0)
C20 = math.sqrt(5.0) / 2.0
C22 = math.sqrt(15.0) / 2.0
EMB_SCALE = 1.14136 * math.exp(2.0) * math.sqrt(10.0)
STEP = 6.0 / 11.0
INV32 = 1.0 / math.sqrt(32.0)


_MESH = plsc.VectorSubcoreMesh(core_axis_name="c", subcore_axis_name="s")


def _make_gather(n_idx, C=80, LR=200):
    EW = n_idx // NW
    NCH = EW // C
    NLCH = N // LR

    @functools.partial(
        pl.kernel,
        out_type=jax.ShapeDtypeStruct((n_idx, DA), jnp.float32),
        mesh=_MESH,
        scratch_types=[
            pltpu.VMEM((C,), jnp.int32),
            pltpu.VMEM((C, DA), jnp.float32),
            pltpu.VMEM_SHARED((N, DA), jnp.float32),
            pltpu.SemaphoreType.DMA,
        ],
    )
    def gather_k(table, idx, out, idx_v, rows_v, tab, sem):
        c = lax.axis_index("c")
        s = lax.axis_index("s")

        def load(r, carry):
            ch = s + NS * r

            @pl.when(ch < NLCH)
            def _():
                pltpu.sync_copy(table.at[pl.ds(ch * LR, LR)],
                                tab.at[pl.ds(ch * LR, LR)])
            return carry

        lax.fori_loop(0, (NLCH + NS - 1) // NS, load, 0)
        plsc.subcore_barrier()

        base = (s * NC + c) * EW

        def body(j, carry):
            off = base + j * C
            pltpu.sync_copy(idx.at[pl.ds(off, C)], idx_v)
            pltpu.async_copy(tab.at[idx_v], rows_v, sem).wait()
            pltpu.sync_copy(rows_v, out.at[pl.ds(off, C)])
            return carry

        lax.fori_loop(0, NCH, body, 0)

    return gather_k


def _make_scatter(C=80, ZR=200):
    EW = E // NW
    NCH = EW // C
    NZCH = N // ZR

    @functools.partial(
        pl.kernel,
        out_type=jax.ShapeDtypeStruct((NC, N, DA), jnp.float32),
        mesh=_MESH,
        scratch_types=[
            pltpu.VMEM((C,), jnp.int32),
            pltpu.VMEM((C, DA), jnp.float32),
            pltpu.VMEM((ZR, DA), jnp.float32),
            pltpu.VMEM_SHARED((N, DA), jnp.float32),
            pltpu.SemaphoreType.DMA,
        ],
    )
    def scatter_k(msg, idx, out, idx_v, msg_v, zb, acc, sem):
        c = lax.axis_index("c")
        s = lax.axis_index("s")

        nvec = ZR * DA // 16

        def zvec(t, carry):
            zb[t // (DA // 16), pl.ds((t % (DA // 16)) * 16, 16)] = jnp.zeros(
                (16,), jnp.float32)
            return carry

        lax.fori_loop(0, nvec, zvec, 0)

        def zacc(r, carry):
            ch = s + NS * r

            @pl.when(ch < NZCH)
            def _():
                pltpu.sync_copy(zb, acc.at[pl.ds(ch * ZR, ZR)])
            return carry

        lax.fori_loop(0, (NZCH + NS - 1) // NS, zacc, 0)
        plsc.subcore_barrier()

        base = (s * NC + c) * EW

        def body(j, carry):
            off = base + j * C
            pltpu.sync_copy(idx.at[pl.ds(off, C)], idx_v)
            pltpu.sync_copy(msg.at[pl.ds(off, C)], msg_v)
            pltpu.sync_copy(msg_v, acc.at[idx_v], add=True)
            return carry

        lax.fori_loop(0, NCH, body, 0)
        plsc.subcore_barrier()

        def wout(r, carry):
            ch = s + NS * r

            @pl.when(ch < NZCH)
            def _():
                pltpu.sync_copy(acc.at[pl.ds(ch * ZR, ZR)],
                                out.at[c, pl.ds(ch * ZR, ZR)])
            return carry

        lax.fori_loop(0, (NZCH + NS - 1) // NS, wout, 0)

    return scatter_k


def _make_posdiff(C=2000):
    EW = E // NW
    NCH = EW // C

    @functools.partial(
        pl.kernel,
        out_type=jax.ShapeDtypeStruct((16 * E,), jnp.float32),
        mesh=_MESH,
        compiler_params=pltpu.CompilerParams(needs_layout_passes=False),
        scratch_types=[
            pltpu.VMEM((4 * N,), jnp.float32),
            pltpu.VMEM((C,), jnp.int32),
            pltpu.VMEM((C,), jnp.int32),
            pltpu.VMEM((16 * C,), jnp.float32),
        ],
    )
    def posdiff_k(pos4, src, dst, out, tab, src_v, dst_v, stage):
        c = lax.axis_index("c")
        s = lax.axis_index("s")
        pltpu.sync_copy(pos4, tab)
        lane = lax.broadcasted_iota(jnp.int32, (16,), 0)
        zero16 = jnp.zeros((16,), jnp.float32)

        def zstage(t, carry):
            stage[pl.ds(t * 16, 16)] = zero16
            return carry

        lax.fori_loop(0, C, zstage, 0)
        base = (s * NC + c) * EW

        def body(j, carry):
            off = base + j * C
            pltpu.sync_copy(src.at[pl.ds(off, C)], src_v)
            pltpu.sync_copy(dst.at[pl.ds(off, C)], dst_v)

            def grp(k, carry2):
                si = src_v[pl.ds(k * 16, 16)] * 4
                di = dst_v[pl.ds(k * 16, 16)] * 4
                ebase = k * 256 + lane * 16
                for d in range(3):
                    ps = plsc.load_gather(tab, [si + d])
                    pd = plsc.load_gather(tab, [di + d])
                    plsc.store_scatter(stage, [ebase + d], ps - pd)
                return carry2

            lax.fori_loop(0, C // 16, grp, 0)
            pltpu.sync_copy(stage, out.at[pl.ds(off * 16, C * 16)])
            return carry

        lax.fori_loop(0, NCH, body, 0)

    return posdiff_k


_posdiff = _make_posdiff()
_gather_e = _make_gather(E)
_scatter = _make_scatter()


TE = 2560
TN = 1000


def _sus(t):
    safe = jnp.where(t > 0, t, 1.0)
    return jnp.where(t > 0, jnp.exp(-1.0 / safe), 0.0)


def _edge_feat_body(dv_ref, attr_ref, feat_ref):
    dv = dv_ref[:, :3]
    ln = jnp.sqrt(jnp.sum(dv * dv, axis=1, keepdims=True) + 1e-12)
    u = dv / ln
    ux, uy, uz = u[:, 0:1], u[:, 1:2], u[:, 2:3]
    sh = jnp.concatenate([
        jnp.ones_like(ux), SQRT3 * ux, SQRT3 * uy, SQRT3 * uz,
        SQRT15 * ux * uy, SQRT15 * uy * uz, C20 * (3.0 * uz * uz - 1.0),
        SQRT15 * ux * uz, C22 * (ux * ux - uy * uy)], axis=1)
    vals = (lax.broadcasted_iota(jnp.int32, (1, 10), 1).astype(jnp.float32)
            + 1.0) * STEP
    diff = (ln - vals) / STEP
    emb = EMB_SCALE * _sus(diff + 1.0) * _sus(1.0 - diff)
    z = jnp.zeros((dv_ref.shape[0], 1), jnp.float32)
    feat_ref[...] = jnp.concatenate(
        [attr_ref[...], sh, z, z, z, emb, z, z, z, z, z, z], axis=1)


def _edge_feat(gdv, edge_attr):
    grid = E // TE
    return pl.pallas_call(
        _edge_feat_body,
        grid=(grid,),
        in_specs=[
            pl.BlockSpec((TE, 16), lambda i: (i, 0)),
            pl.BlockSpec((TE, 4), lambda i: (i, 0)),
        ],
        out_specs=pl.BlockSpec((TE, FEAT), lambda i: (i, 0)),
        out_shape=jax.ShapeDtypeStruct((E, FEAT), jnp.float32),
    )(gdv, edge_attr)


def _combine1_body(feat_ref, ga_ref, weaa_ref, wf1_ref, wf2a_ref, msga_ref):
    f = feat_ref[...]
    gl = jax.nn.gelu(jnp.dot(f, wf1_ref[...],
                             preferred_element_type=jnp.float32))
    wea = jnp.dot(gl, wf2a_ref[...], preferred_element_type=jnp.float32)
    eaa = jnp.dot(f, weaa_ref[...], preferred_element_type=jnp.float32)
    msga_ref[...] = (ga_ref[...] + eaa) * wea


def _combine1(feat, ga, weaa, wf1, wf2a):
    grid = E // TE
    return pl.pallas_call(
        _combine1_body,
        grid=(grid,),
        in_specs=[
            pl.BlockSpec((TE, FEAT), lambda i: (i, 0)),
            pl.BlockSpec((TE, DA), lambda i: (i, 0)),
            pl.BlockSpec((FEAT, DA), lambda i: (0, 0)),
            pl.BlockSpec((FEAT, 100), lambda i: (0, 0)),
            pl.BlockSpec((100, DA), lambda i: (0, 0)),
        ],
        out_specs=pl.BlockSpec((TE, DA), lambda i: (i, 0)),
        out_shape=jax.ShapeDtypeStruct((E, DA), jnp.float32),
    )(feat, ga, weaa, wf1, wf2a)


def _pre0_body(x_ref, wa_ref, wb_ref, hwa_ref, hwb_ref):
    x = x_ref[...]
    hwa_ref[...] = jnp.dot(x, wa_ref[...], preferred_element_type=jnp.float32)
    hwb_ref[...] = jnp.dot(x, wb_ref[...], preferred_element_type=jnp.float32)


def _pre0(x, wa, wb):
    return pl.pallas_call(
        _pre0_body,
        grid=(N // TN,),
        in_specs=[
            pl.BlockSpec((TN, 8), lambda i: (i, 0)),
            pl.BlockSpec((8, DA), lambda i: (0, 0)),
            pl.BlockSpec((8, DA), lambda i: (0, 0)),
        ],
        out_specs=[
            pl.BlockSpec((TN, DA), lambda i: (i, 0)),
            pl.BlockSpec((TN, DA), lambda i: (i, 0)),
        ],
        out_shape=[
            jax.ShapeDtypeStruct((N, DA), jnp.float32),
            jax.ShapeDtypeStruct((N, DA), jnp.float32),
        ],
    )(x, wa, wb)


def _upd_body(aa0_ref, aa1_ref, ab0_ref, ab1_ref, h_ref, na_ref, ws_ref,
              wta_ref, wtb_ref, hout_ref, hwa_ref, hwb_ref):
    agga = (aa0_ref[0] + aa1_ref[0]) * INV32
    aggb = ((ab0_ref[0] + ab1_ref[0]) * INV32)[:, :DH - DA]
    agg = jnp.concatenate([agga, aggb], axis=1)
    hn = agg + jnp.dot(h_ref[...], ws_ref[...],
                       preferred_element_type=jnp.float32)
    hn = jax.nn.gelu(hn * na_ref[...])
    hout_ref[...] = hn
    hwa_ref[...] = jnp.dot(hn, wta_ref[...], preferred_element_type=jnp.float32)
    hwb_ref[...] = jnp.dot(hn, wtb_ref[...], preferred_element_type=jnp.float32)


def _upd(agga, aggb, h, node_attr, wself, wta, wtb, di):
    return pl.pallas_call(
        _upd_body,
        grid=(N // TN,),
        in_specs=[
            pl.BlockSpec((1, TN, DA), lambda i: (0, i, 0)),
            pl.BlockSpec((1, TN, DA), lambda i: (1, i, 0)),
            pl.BlockSpec((1, TN, DA), lambda i: (0, i, 0)),
            pl.BlockSpec((1, TN, DA), lambda i: (1, i, 0)),
            pl.BlockSpec((TN, di), lambda i: (i, 0)),
            pl.BlockSpec((TN, 1), lambda i: (i, 0)),
            pl.BlockSpec((di, DH), lambda i: (0, 0)),
            pl.BlockSpec((DH, DA), lambda i: (0, 0)),
            pl.BlockSpec((DH, DA), lambda i: (0, 0)),
        ],
        out_specs=[
            pl.BlockSpec((TN, DH), lambda i: (i, 0)),
            pl.BlockSpec((TN, DA), lambda i: (i, 0)),
            pl.BlockSpec((TN, DA), lambda i: (i, 0)),
        ],
        out_shape=[
            jax.ShapeDtypeStruct((N, DH), jnp.float32),
            jax.ShapeDtypeStruct((N, DA), jnp.float32),
            jax.ShapeDtypeStruct((N, DA), jnp.float32),
        ],
    )(agga, agga, aggb, aggb, h, node_attr, wself, wta, wtb)


def _fin_body(aa0_ref, aa1_ref, h_ref, na_ref, ws_ref, out_ref):
    i = pl.program_id(0)
    agg = (aa0_ref[0][:, :3] + aa1_ref[0][:, :3]) * INV32
    hv = (agg + jnp.dot(h_ref[...], ws_ref[...],
                        preferred_element_type=jnp.float32)) * na_ref[...]
    part = jnp.sum(hv, axis=0, keepdims=True) * 0.01

    @pl.when(i == 0)
    def _():
        out_ref[...] = part

    @pl.when(i > 0)
    def _():
        out_ref[...] += part


def _fin(agga, h, node_attr, wself3):
    return pl.pallas_call(
        _fin_body,
        grid=(N // TN,),
        in_specs=[
            pl.BlockSpec((1, TN, DA), lambda i: (0, i, 0)),
            pl.BlockSpec((1, TN, DA), lambda i: (1, i, 0)),
            pl.BlockSpec((TN, DH), lambda i: (i, 0)),
            pl.BlockSpec((TN, 1), lambda i: (i, 0)),
            pl.BlockSpec((DH, 3), lambda i: (0, 0)),
        ],
        out_specs=pl.BlockSpec((1, 3), lambda i: (0, 0)),
        out_shape=jax.ShapeDtypeStruct((1, 3), jnp.float32),
    )(agga, agga, h, node_attr, wself3)


def _pad_cols(a, w):
    return jnp.pad(a, ((0, 0), (0, w - a.shape[1])))


def kernel(pos, x, node_attr, edge_attr, edge_src, edge_dst, batch,
           W_msg_0, W_fc1_0, W_fc2_0, W_self_0,
           W_msg_1, W_fc1_1, W_fc2_1, W_self_1,
           W_msg_2, W_fc1_2, W_fc2_2, W_self_2,
           W_msg_3, W_fc1_3, W_fc2_3, W_self_3):
    Wm = [W_msg_0, W_msg_1, W_msg_2, W_msg_3]
    Wf1 = [W_fc1_0, W_fc1_1, W_fc1_2, W_fc1_3]
    Wf2 = [W_fc2_0, W_fc2_1, W_fc2_2, W_fc2_3]
    Ws = [W_self_0, W_self_1, W_self_2, W_self_3]
    dims = [8, DH, DH, DH]

    wtopa, wtopb = [], []
    weaa_p, weab_p, wf1_p, wf2a_p, wf2b_p = [], [], [], [], []
    for i in range(4):
        di = dims[i]
        wtop_i = Wm[i][:di]
        wea_i = Wm[i][di:]
        do = Wm[i].shape[1]
        wtopa.append(_pad_cols(wtop_i[:, :min(do, DA)], DA))
        wtopb.append(_pad_cols(wtop_i[:, min(do, DA):], DA))
        wea_full = jnp.concatenate(
            [wea_i, jnp.zeros((FEAT - 13, do), jnp.float32)], axis=0)
        weaa_p.append(_pad_cols(wea_full[:, :min(do, DA)], DA))
        weab_p.append(_pad_cols(wea_full[:, min(do, DA):], DA))
        wf1_p.append(jnp.concatenate([
            jnp.zeros((16, 100), jnp.float32), Wf1[i],
            jnp.zeros((6, 100), jnp.float32)], axis=0))
        wf2a_p.append(_pad_cols(Wf2[i][:, :min(do, DA)], DA))
        wf2b_p.append(_pad_cols(Wf2[i][:, min(do, DA):], DA))

    pos4 = _pad_cols(pos, 4).reshape(-1)
    gdv = _posdiff(pos4, edge_src, edge_dst).reshape(E, 16)
    feat = _edge_feat(gdv, edge_attr)

    hwa, hwb = _pre0(x, wtopa[0], wtopb[0])
    h = x
    for i in range(3):
        ga = _gather_e(hwa, edge_src)
        srcb, _ = lax.optimization_barrier((edge_src, ga))
        gb = _gather_e(hwb, srcb)
        msga = _combine1(feat, ga, weaa_p[i], wf1_p[i], wf2a_p[i])
        msga2, _ = lax.optimization_barrier((msga, gb))
        agga = _scatter(msga2, edge_dst)
        msgb = _combine1(feat, gb, weab_p[i], wf1_p[i], wf2b_p[i])
        dst2, _ = lax.optimization_barrier((edge_dst, agga))
        aggb = _scatter(msgb, dst2)
        h, hwa, hwb = _upd(agga, aggb, h, node_attr, Ws[i],
                           wtopa[i + 1], wtopb[i + 1], dims[i])
    ga = _gather_e(hwa, edge_src)
    msga = _combine1(feat, ga, weaa_p[3], wf1_p[3], wf2a_p[3])
    agga = _scatter(msga, edge_dst)
    return _fin(agga, h, node_attr, Ws[3])
